# baseline (device time: 15352 ns/iter reference)
import jax
import jax.numpy as jnp
from jax import lax
from jax.experimental import pallas as pl
from jax.experimental.pallas import tpu as pltpu

C = 8


def kernel(x):
    m, n = x.shape
    half = m // 2
    rc = half // C
    rc2 = rc // 2

    def body(
        x_ref,
        out_ref,
        y_send,
        y_recv,
        s_buf,
        x_recv,
        z_recv,
        y_ssem,
        y_rsem,
        x_ssem,
        x_rsem,
        z_ssem,
        z_rsem,
        xz_ready,
    ):
        my_x = lax.axis_index("x")
        my_y = lax.axis_index("y")
        my_z = lax.axis_index("z")
        h = (my_x + my_y + my_z) % 2
        oh = 1 - h
        y_peer = (my_x, 1 - my_y, my_z)
        x_peer = (1 - my_x, my_y, my_z)
        z_peer = (my_x, my_y, 1 - my_z)

        barrier_sem = pltpu.get_barrier_semaphore()
        pl.semaphore_signal(
            barrier_sem, inc=1,
            device_id=y_peer, device_id_type=pl.DeviceIdType.MESH,
        )
        for peer in (x_peer, z_peer):
            pl.semaphore_signal(
                xz_ready, inc=1,
                device_id=peer, device_id_type=pl.DeviceIdType.MESH,
            )
        y_send[...] = x_ref[pl.ds(oh * half, half), :].astype(jnp.bfloat16)
        s_buf[...] = x_ref[pl.ds(h * half, half), :].astype(jnp.bfloat16)

        pl.semaphore_wait(barrier_sem, 1)

        def mk_y(c):
            sl = pl.ds(c * rc, rc)
            return pltpu.make_async_remote_copy(
                src_ref=y_send.at[sl],
                dst_ref=y_recv.at[sl],
                send_sem=y_ssem.at[c],
                recv_sem=y_rsem.at[c],
                device_id=y_peer,
                device_id_type=pl.DeviceIdType.MESH,
            )

        rdma_y = [mk_y(c) for c in range(C)]

        for c in range(C):
            rdma_y[c].start()

        def mk_xz(c, recv_buf, ssem, rsem, peer, piece):
            return pltpu.make_async_remote_copy(
                src_ref=s_buf.at[pl.ds(c * rc + piece * rc2, rc2)],
                dst_ref=recv_buf.at[pl.ds(c * rc2, rc2)],
                send_sem=ssem.at[c],
                recv_sem=rsem.at[c],
                device_id=peer,
                device_id_type=pl.DeviceIdType.MESH,
            )

        rdma_x = [mk_xz(c, x_recv, x_ssem, x_rsem, x_peer, 0) for c in range(C)]
        rdma_z = [mk_xz(c, z_recv, z_ssem, z_rsem, z_peer, 1) for c in range(C)]

        for c in range(C):
            rdma_y[c].wait_recv()
            s_buf[pl.ds(c * rc, rc), :] = (
                s_buf[pl.ds(c * rc, rc), :] + y_recv[pl.ds(c * rc, rc), :]
            )
            if c == 0:
                pl.semaphore_wait(xz_ready, 2)
            rdma_x[c].start()
            rdma_z[c].start()
            out_ref[pl.ds(h * half + c * rc, rc), :] = s_buf[pl.ds(c * rc, rc), :]

        for c in range(C):
            rdma_x[c].wait_recv()
            out_ref[pl.ds(oh * half + c * rc, rc2), :] = x_recv[
                pl.ds(c * rc2, rc2), :
            ]
            rdma_z[c].wait_recv()
            out_ref[pl.ds(oh * half + c * rc + rc2, rc2), :] = z_recv[
                pl.ds(c * rc2, rc2), :
            ]

        for c in range(C):
            rdma_y[c].wait_send()
            rdma_x[c].wait_send()
            rdma_z[c].wait_send()

    return pl.pallas_call(
        body,
        out_shape=jax.ShapeDtypeStruct((m, n), jnp.bfloat16),
        in_specs=[pl.BlockSpec(memory_space=pltpu.VMEM)],
        out_specs=pl.BlockSpec(memory_space=pltpu.VMEM),
        scratch_shapes=[
            pltpu.VMEM((half, n), jnp.bfloat16),
            pltpu.VMEM((half, n), jnp.bfloat16),
            pltpu.VMEM((half, n), jnp.bfloat16),
            pltpu.VMEM((half // 2, n), jnp.bfloat16),
            pltpu.VMEM((half // 2, n), jnp.bfloat16),
            pltpu.SemaphoreType.DMA((C,)),
            pltpu.SemaphoreType.DMA((C,)),
            pltpu.SemaphoreType.DMA((C,)),
            pltpu.SemaphoreType.DMA((C,)),
            pltpu.SemaphoreType.DMA((C,)),
            pltpu.SemaphoreType.DMA((C,)),
            pltpu.SemaphoreType.REGULAR,
        ],
        compiler_params=pltpu.CompilerParams(collective_id=0),
    )(x)


# device time: 15183 ns/iter; 1.0111x vs baseline; 1.0111x over previous
import jax
import jax.numpy as jnp
from jax import lax
from jax.experimental import pallas as pl
from jax.experimental.pallas import tpu as pltpu

SIZES = (128, 96, 64, 64, 48, 48, 32, 32)
OFFS = tuple(sum(SIZES[:i]) for i in range(len(SIZES)))
C = len(SIZES)


def kernel(x):
    m, n = x.shape
    half = m // 2
    assert sum(SIZES) == half

    def body(
        x_ref,
        out_ref,
        y_send,
        y_recv,
        s_buf,
        x_recv,
        z_recv,
        y_ssem,
        y_rsem,
        x_ssem,
        x_rsem,
        z_ssem,
        z_rsem,
        xz_ready,
    ):
        my_x = lax.axis_index("x")
        my_y = lax.axis_index("y")
        my_z = lax.axis_index("z")
        h = (my_x + my_y + my_z) % 2
        oh = 1 - h
        y_peer = (my_x, 1 - my_y, my_z)
        x_peer = (1 - my_x, my_y, my_z)
        z_peer = (my_x, my_y, 1 - my_z)

        barrier_sem = pltpu.get_barrier_semaphore()
        pl.semaphore_signal(
            barrier_sem, inc=1,
            device_id=y_peer, device_id_type=pl.DeviceIdType.MESH,
        )
        for peer in (x_peer, z_peer):
            pl.semaphore_signal(
                xz_ready, inc=1,
                device_id=peer, device_id_type=pl.DeviceIdType.MESH,
            )
        y_send[...] = x_ref[pl.ds(oh * half, half), :].astype(jnp.bfloat16)
        s_buf[...] = x_ref[pl.ds(h * half, half), :].astype(jnp.bfloat16)

        pl.semaphore_wait(barrier_sem, 1)

        def mk_y(c):
            sl = pl.ds(OFFS[c], SIZES[c])
            return pltpu.make_async_remote_copy(
                src_ref=y_send.at[sl],
                dst_ref=y_recv.at[sl],
                send_sem=y_ssem.at[c],
                recv_sem=y_rsem.at[c],
                device_id=y_peer,
                device_id_type=pl.DeviceIdType.MESH,
            )

        rdma_y = [mk_y(c) for c in range(C)]

        for c in range(C):
            rdma_y[c].start()

        def mk_xz(c, recv_buf, ssem, rsem, peer, piece):
            sz2 = SIZES[c] // 2
            return pltpu.make_async_remote_copy(
                src_ref=s_buf.at[pl.ds(OFFS[c] + piece * sz2, sz2)],
                dst_ref=recv_buf.at[pl.ds(OFFS[c] // 2, sz2)],
                send_sem=ssem.at[c],
                recv_sem=rsem.at[c],
                device_id=peer,
                device_id_type=pl.DeviceIdType.MESH,
            )

        rdma_x = [mk_xz(c, x_recv, x_ssem, x_rsem, x_peer, 0) for c in range(C)]
        rdma_z = [mk_xz(c, z_recv, z_ssem, z_rsem, z_peer, 1) for c in range(C)]

        for c in range(C):
            off, sz = OFFS[c], SIZES[c]
            rdma_y[c].wait_recv()
            s_buf[pl.ds(off, sz), :] = (
                s_buf[pl.ds(off, sz), :] + y_recv[pl.ds(off, sz), :]
            )
            if c == 0:
                pl.semaphore_wait(xz_ready, 2)
            rdma_x[c].start()
            rdma_z[c].start()
            out_ref[pl.ds(h * half + off, sz), :] = s_buf[pl.ds(off, sz), :]

        for c in range(C):
            off, sz2 = OFFS[c], SIZES[c] // 2
            rdma_x[c].wait_recv()
            out_ref[pl.ds(oh * half + off, sz2), :] = x_recv[
                pl.ds(off // 2, sz2), :
            ]
            rdma_z[c].wait_recv()
            out_ref[pl.ds(oh * half + off + sz2, sz2), :] = z_recv[
                pl.ds(off // 2, sz2), :
            ]

        for c in range(C):
            rdma_y[c].wait_send()
            rdma_x[c].wait_send()
            rdma_z[c].wait_send()

    return pl.pallas_call(
        body,
        out_shape=jax.ShapeDtypeStruct((m, n), jnp.bfloat16),
        in_specs=[pl.BlockSpec(memory_space=pltpu.VMEM)],
        out_specs=pl.BlockSpec(memory_space=pltpu.VMEM),
        scratch_shapes=[
            pltpu.VMEM((half, n), jnp.bfloat16),
            pltpu.VMEM((half, n), jnp.bfloat16),
            pltpu.VMEM((half, n), jnp.bfloat16),
            pltpu.VMEM((half // 2, n), jnp.bfloat16),
            pltpu.VMEM((half // 2, n), jnp.bfloat16),
            pltpu.SemaphoreType.DMA((C,)),
            pltpu.SemaphoreType.DMA((C,)),
            pltpu.SemaphoreType.DMA((C,)),
            pltpu.SemaphoreType.DMA((C,)),
            pltpu.SemaphoreType.DMA((C,)),
            pltpu.SemaphoreType.DMA((C,)),
            pltpu.SemaphoreType.REGULAR,
        ],
        compiler_params=pltpu.CompilerParams(collective_id=0),
    )(x)
